# baseline (speedup 1.0000x reference)
import functools

import jax
import jax.numpy as jnp
from jax import lax
from jax.experimental import pallas as pl
from jax.experimental.pallas import tpu as pltpu
from jax.experimental.pallas import tpu_sc as plsc

N = 10000
E = 320000
D = 128
NC = 2
NS = 16
NPAD = 10240
NR = NPAD // 128
CH = 32
EPW = E // (NC * NS)
NCHUNK = EPW // CH
CHT = EPW - NCHUNK * CH


def _sc_body(xs_h, asn_h, adn_h, ae_h, src_h, dst_h,
             outp_h, denp_h,
             asn_t, adn_t, den2d, sbuf, dbuf, aebuf, exbuf, rows, iota_r,
             stail, dtail, aetail,
             outacc, densh,
             gsem0, gsem1, gsem2, ssem0, ssem1, ssem2,
             tsem0, tsem1, tsem2, tsem3, tsem4, tsem5, dsem):
  c = lax.axis_index("c")
  s = lax.axis_index("s")
  base = (c * NS + s) * EPW
  gsems = (gsem0, gsem1, gsem2)
  ssems = (ssem0, ssem1, ssem2)
  tsems = (tsem0, tsem1, tsem2, tsem3, tsem4, tsem5)

  z16 = jnp.zeros((16,), jnp.float32)

  def _zero(i, _):
    for j in range(8):
      asn_t[i, pl.ds(j * 16, 16)] = z16
      den2d[i, pl.ds(j * 16, 16)] = z16
    return ()
  lax.fori_loop(0, NR, _zero, ())

  for k in range(NPAD // NS // NR):
    pltpu.async_copy(asn_t, outacc.at[pl.ds((s * 8 + k) * NR, NR)],
                     gsems[k % 2])
  @pl.when(s < NR // 8)
  def _():
    pltpu.async_copy(asn_t.at[pl.ds(0, 8)], densh.at[pl.ds(s * 8, 8)],
                     gsems[2])
  for k in range(NPAD // NS // NR):
    pltpu.make_async_copy(asn_t, outacc.at[pl.ds((s * 8 + k) * NR, NR)],
                          gsems[k % 2]).wait()
  @pl.when(s < NR // 8)
  def _():
    pltpu.make_async_copy(asn_t.at[pl.ds(0, 8)], densh.at[pl.ds(s * 8, 8)],
                          gsems[2]).wait()

  pltpu.sync_copy(asn_h, asn_t)
  pltpu.sync_copy(adn_h, adn_t)

  ii = lax.iota(jnp.int32, 16)
  for i in range(NR // 16):
    iota_r[0, pl.ds(i * 16, 16)] = ii + (i * 16)

  plsc.subcore_barrier()

  def stage(g, t):
    off = base + g * CH
    pltpu.async_copy(src_h.at[pl.ds(off, CH)], sbuf.at[t], tsems[t])
    pltpu.async_copy(dst_h.at[pl.ds(off, CH)], dbuf.at[t], tsems[t])
    pltpu.async_copy(ae_h.at[pl.ds(off, CH)], aebuf.at[t], tsems[t])

  def wait_stage(g, t):
    off = base + g * CH
    pltpu.make_async_copy(src_h.at[pl.ds(off, CH)], sbuf.at[t], tsems[t]).wait()
    pltpu.make_async_copy(dst_h.at[pl.ds(off, CH)], dbuf.at[t], tsems[t]).wait()
    pltpu.make_async_copy(ae_h.at[pl.ds(off, CH)], aebuf.at[t], tsems[t]).wait()

  def gather(t, b):
    pltpu.async_copy(xs_h.at[sbuf.at[t]], rows.at[b], gsems[b])

  def wait_gather(t, b):
    pltpu.make_async_copy(xs_h.at[sbuf.at[t]], rows.at[b], gsems[b]).wait()

  def wait_scatter(t, b):
    pltpu.make_async_copy(rows.at[b], outacc.at[dbuf.at[t]], ssems[b]).wait()

  def ex_compute(t):
    for i in range(CH // 16):
      s16 = sbuf[t, pl.ds(i * 16, 16)]
      d16 = dbuf[t, pl.ds(i * 16, 16)]
      av = (plsc.load_gather(asn_t, [s16 >> 7, s16 & 127])
            + plsc.load_gather(adn_t, [d16 >> 7, d16 & 127])
            + aebuf[t, pl.ds(i * 16, 16)])
      av = jnp.maximum(av, 0.2 * av)
      ex = jnp.exp(av)
      exbuf[0, pl.ds(i * 16, 16)] = ex
      plsc.addupdate_scatter(den2d, [d16 >> 7, d16 & 127], ex)

  def scale_scatter(t, b):
    def _srow(i, _):
      ex16 = exbuf[0, pl.ds(i * 16, 16)]
      for k in range(16):
        cv = jnp.full((16,), ex16[k], jnp.float32)
        e = i * 16 + k
        for j in range(8):
          rows[b, e, pl.ds(j * 16, 16)] = rows[b, e, pl.ds(j * 16, 16)] * cv
      return ()
    lax.fori_loop(0, CH // 16, _srow, ())

    pltpu.async_copy(rows.at[b], outacc.at[dbuf.at[t]], ssems[b], add=True)

  for g0 in range(4):
    stage(g0, g0)
  wait_stage(0, 0)
  gather(0, 0)
  wait_stage(1, 1)
  gather(1, 1)

  def chunk_body(g, u):
    t = u % 6
    b = u % 3
    ex_compute(t)

    @pl.when(g >= 1)
    def _():
      wait_scatter((u - 1) % 6, (u - 1) % 3)

    @pl.when(g + 2 < NCHUNK)
    def _():
      wait_stage(g + 2, (u + 2) % 6)
      gather((u + 2) % 6, (u + 2) % 3)

    @pl.when(g + 4 < NCHUNK)
    def _():
      stage(g + 4, (u + 4) % 6)
    wait_gather(b, b)
    scale_scatter(t, b)

  def six(p, _):
    for u in range(6):
      chunk_body(6 * p + u, u)
    return ()
  lax.fori_loop(0, NCHUNK // 6, six, ())

  wait_scatter((NCHUNK - 1) % 6, (NCHUNK - 1) % 3)

  offt = base + NCHUNK * CH
  pltpu.sync_copy(src_h.at[pl.ds(offt, CHT)], stail.at[0])
  pltpu.sync_copy(dst_h.at[pl.ds(offt, CHT)], dtail.at[0])
  pltpu.sync_copy(ae_h.at[pl.ds(offt, CHT)], aetail.at[0])
  pltpu.async_copy(xs_h.at[stail.at[0]], rows.at[0, pl.ds(0, CHT)],
                   gsems[0]).wait()
  for i in range(CHT // 16):
    s16 = stail[0, pl.ds(i * 16, 16)]
    d16 = dtail[0, pl.ds(i * 16, 16)]
    av = (plsc.load_gather(asn_t, [s16 >> 7, s16 & 127])
          + plsc.load_gather(adn_t, [d16 >> 7, d16 & 127])
          + aetail[0, pl.ds(i * 16, 16)])
    av = jnp.maximum(av, 0.2 * av)
    ex = jnp.exp(av)
    plsc.addupdate_scatter(den2d, [d16 >> 7, d16 & 127], ex)
    for k in range(16):
      cv = jnp.full((16,), ex[k], jnp.float32)
      e = i * 16 + k
      for j in range(8):
        rows[0, e, pl.ds(j * 16, 16)] = rows[0, e, pl.ds(j * 16, 16)] * cv
  pltpu.async_copy(rows.at[0, pl.ds(0, CHT)], outacc.at[dtail.at[0]],
                   ssems[0], add=True).wait()

  plsc.subcore_barrier()

  pltpu.async_copy(den2d, densh.at[iota_r.at[0]], dsem, add=True).wait()
  plsc.subcore_barrier()

  rows_per_tile = NPAD // NS
  pltpu.sync_copy(outacc.at[pl.ds(s * rows_per_tile, rows_per_tile)],
                  outp_h.at[c, pl.ds(s * rows_per_tile, rows_per_tile)])
  @pl.when(s < NR // 8)
  def _():
    pltpu.sync_copy(densh.at[pl.ds(s * 8, 8)],
                    denp_h.at[c, pl.ds(s * 8, 8)])


@functools.cache
def _sc_edge_kernel():
  return pl.kernel(
    _sc_body,
    out_type=[
        jax.ShapeDtypeStruct((NC, NPAD, D), jnp.float32),
        jax.ShapeDtypeStruct((NC, NR, 128), jnp.float32),
    ],
    mesh=plsc.VectorSubcoreMesh(core_axis_name="c", subcore_axis_name="s",
                                num_cores=NC, num_subcores=NS),
    compiler_params=pltpu.CompilerParams(needs_layout_passes=False),
    scratch_types=[
        pltpu.VMEM((NR, 128), jnp.float32),
        pltpu.VMEM((NR, 128), jnp.float32),
        pltpu.VMEM((NR, 128), jnp.float32),
        pltpu.VMEM((6, CH), jnp.int32),
        pltpu.VMEM((6, CH), jnp.int32),
        pltpu.VMEM((6, CH), jnp.float32),
        pltpu.VMEM((1, CH), jnp.float32),
        pltpu.VMEM((3, CH, D), jnp.float32),
        pltpu.VMEM((1, NR), jnp.int32),
        pltpu.VMEM((1, CHT), jnp.int32),
        pltpu.VMEM((1, CHT), jnp.int32),
        pltpu.VMEM((1, CHT), jnp.float32),
        pltpu.VMEM_SHARED((NPAD, D), jnp.float32),
        pltpu.VMEM_SHARED((NR, 128), jnp.float32),
        pltpu.SemaphoreType.DMA,
        pltpu.SemaphoreType.DMA,
        pltpu.SemaphoreType.DMA,
        pltpu.SemaphoreType.DMA,
        pltpu.SemaphoreType.DMA,
        pltpu.SemaphoreType.DMA,
        pltpu.SemaphoreType.DMA,
        pltpu.SemaphoreType.DMA,
        pltpu.SemaphoreType.DMA,
        pltpu.SemaphoreType.DMA,
        pltpu.SemaphoreType.DMA,
        pltpu.SemaphoreType.DMA,
        pltpu.SemaphoreType.DMA,
    ],
  )


def _sc_edge(*args):
  return _sc_edge_kernel()(*args)


_NB = 10
_BN = NPAD // _NB
_BE = 8192
_EB = -(-E // _BE)


def _full(shape):
  return pl.BlockSpec(shape, lambda i: tuple(0 for _ in shape))


def _rows(bs, width=None):
  if width is None:
    return pl.BlockSpec((bs,), lambda i: (i,))
  return pl.BlockSpec((bs, width), lambda i: (i, 0))


def _pre_body(x_r, ws_r, as_r, wd_r, ad_r, xs_r, asn_r, adn_r):
  x = x_r[...]
  xs = jnp.dot(x, ws_r[...], preferred_element_type=jnp.float32)
  xs_r[...] = xs
  asn_r[...] = jnp.sum(xs * as_r[...], axis=1)
  xd = jnp.dot(x, wd_r[...], preferred_element_type=jnp.float32)
  adn_r[...] = jnp.sum(xd * ad_r[...], axis=1)


def _tc_pre(x, ws, a_s, wd, a_d):
  return pl.pallas_call(
      _pre_body,
      grid=(_NB,),
      in_specs=[_rows(_BN, D), _full((D, D)), _full((1, D)),
                _full((D, D)), _full((1, D))],
      out_specs=[_rows(_BN, D), _rows(_BN), _rows(_BN)],
      out_shape=[jax.ShapeDtypeStruct((NPAD, D), jnp.float32),
                 jax.ShapeDtypeStruct((NPAD,), jnp.float32),
                 jax.ShapeDtypeStruct((NPAD,), jnp.float32)],
  )(x, ws, a_s.reshape(1, D), wd, a_d.reshape(1, D))


def _ae_body(ea_r, we1_r, ae1_r, we2_r, ae2_r, o1_r, o2_r):
  ea = ea_r[...]
  v1 = jnp.sum(we1_r[...] * ae1_r[...], axis=1)
  v2 = jnp.sum(we2_r[...] * ae2_r[...], axis=1)
  o1_r[...] = jnp.sum(ea * v1[None, :], axis=1)
  o2_r[...] = jnp.sum(ea * v2[None, :], axis=1)


def _tc_ae(edge_attr, we1, ae1, we2, ae2):
  de = edge_attr.shape[1]
  return pl.pallas_call(
      _ae_body,
      grid=(_EB,),
      in_specs=[pl.BlockSpec((_BE, de), lambda i: (i, 0)),
                _full((de, D)), _full((1, D)),
                _full((de, D)), _full((1, D))],
      out_specs=[_rows(_BE), _rows(_BE)],
      out_shape=[jax.ShapeDtypeStruct((E,), jnp.float32),
                 jax.ShapeDtypeStruct((E,), jnp.float32)],
  )(edge_attr, we1, ae1.reshape(1, D), we2, ae2.reshape(1, D))


def _mid_body(op_r, dp_r, x_r, wl_r, bl_r, b_r,
              ws2_r, as2_r, wd2_r, ad2_r,
              h_r, xs2_r, asn2_r, adn2_r):
  den = dp_r[0] + dp_r[1]
  gat = (op_r[0] + op_r[1]) * (1.0 / (den + 1e-16))[:, None] + b_r[...]
  lin = jnp.dot(x_r[...], wl_r[...], preferred_element_type=jnp.float32)
  h = jnp.maximum(gat + lin + bl_r[...], 0.0)
  h_r[...] = h
  xs2 = jnp.dot(h, ws2_r[...], preferred_element_type=jnp.float32)
  xs2_r[...] = xs2
  asn2_r[...] = jnp.sum(xs2 * as2_r[...], axis=1)
  xd2 = jnp.dot(h, wd2_r[...], preferred_element_type=jnp.float32)
  adn2_r[...] = jnp.sum(xd2 * ad2_r[...], axis=1)


def _tc_mid(outp, denp, x, wl, bl, b, ws2, as2, wd2, ad2):
  return pl.pallas_call(
      _mid_body,
      grid=(_NB,),
      in_specs=[pl.BlockSpec((NC, _BN, D), lambda i: (0, i, 0)),
                pl.BlockSpec((NC, _BN), lambda i: (0, i)),
                _rows(_BN, D), _full((D, D)), _full((1, D)), _full((1, D)),
                _full((D, D)), _full((1, D)), _full((D, D)), _full((1, D))],
      out_specs=[_rows(_BN, D), _rows(_BN, D), _rows(_BN), _rows(_BN)],
      out_shape=[jax.ShapeDtypeStruct((NPAD, D), jnp.float32),
                 jax.ShapeDtypeStruct((NPAD, D), jnp.float32),
                 jax.ShapeDtypeStruct((NPAD,), jnp.float32),
                 jax.ShapeDtypeStruct((NPAD,), jnp.float32)],
  )(outp, denp, x, wl, bl.reshape(1, D), b.reshape(1, D),
    ws2, as2.reshape(1, D), wd2, ad2.reshape(1, D))


def _post_body(op_r, dp_r, h_r, wl_r, bl_r, b_r, o_r):
  den = dp_r[0] + dp_r[1]
  gat = (op_r[0] + op_r[1]) * (1.0 / (den + 1e-16))[:, None] + b_r[...]
  lin = jnp.dot(h_r[...], wl_r[...], preferred_element_type=jnp.float32)
  o_r[...] = gat + lin + bl_r[...]


def _tc_post(outp, denp, h, wl, bl, b):
  return pl.pallas_call(
      _post_body,
      grid=(_NB,),
      in_specs=[pl.BlockSpec((NC, _BN, D), lambda i: (0, i, 0)),
                pl.BlockSpec((NC, _BN), lambda i: (0, i)),
                _rows(_BN, D), _full((D, D)), _full((1, D)), _full((1, D))],
      out_specs=_rows(_BN, D),
      out_shape=jax.ShapeDtypeStruct((N, D), jnp.float32),
  )(outp, denp, h, wl, bl.reshape(1, D), b.reshape(1, D))


@jax.jit
def kernel(x, edge_index, edge_attr,
           W_src1, W_dst1, att_src1, att_dst1, W_edge1, att_edge1, b1, Wl1, bl1,
           W_src2, W_dst2, att_src2, att_dst2, W_edge2, att_edge2, b2, Wl2, bl2):
  src = edge_index[0]
  dst = edge_index[1]

  ae1, ae2 = _tc_ae(edge_attr, W_edge1, att_edge1, W_edge2, att_edge2)

  xs1, asn1, adn1 = _tc_pre(x, W_src1, att_src1, W_dst1, att_dst1)
  outp1, denp1 = _sc_edge(xs1, asn1.reshape(NR, 128), adn1.reshape(NR, 128),
                          ae1, src, dst)
  h, xs2, asn2, adn2 = _tc_mid(
      outp1, denp1.reshape(NC, NPAD), x, Wl1, bl1, b1,
      W_src2, att_src2, W_dst2, att_dst2)

  outp2, denp2 = _sc_edge(xs2, asn2.reshape(NR, 128), adn2.reshape(NR, 128),
                          ae2, src, dst)
  return _tc_post(outp2, denp2.reshape(NC, NPAD), h, Wl2, bl2, b2)


# speedup vs baseline: 1.8833x; 1.8833x over previous
import functools

import jax
import jax.numpy as jnp
from jax import lax
from jax.experimental import pallas as pl
from jax.experimental.pallas import tpu as pltpu
from jax.experimental.pallas import tpu_sc as plsc

N = 10000
E = 320000
D = 128
NC = 2
NS = 16
NPAD = 10240
NR = NPAD // 128
CH = 32
EPW = E // (NC * NS)
NCHUNK = EPW // CH
CHT = EPW - NCHUNK * CH


def _sc_body(xs_h, asn_h, adn_h, ae_h, src_h, dst_h,
             outp_h, denp_h,
             asn_t, adn_t, den2d, sbuf, dbuf, aebuf, exbuf, rows, iota_r,
             stail, dtail, aetail,
             outacc, densh,
             gsem0, gsem1, gsem2, ssem0, ssem1, ssem2,
             tsem0, tsem1, tsem2, tsem3, tsem4, tsem5, dsem):
  c = lax.axis_index("c")
  s = lax.axis_index("s")
  base = (c * NS + s) * EPW
  gsems = (gsem0, gsem1, gsem2)
  ssems = (ssem0, ssem1, ssem2)
  tsems = (tsem0, tsem1, tsem2, tsem3, tsem4, tsem5)

  z16 = jnp.zeros((16,), jnp.float32)

  def _zero(i, _):
    for j in range(8):
      asn_t[i, pl.ds(j * 16, 16)] = z16
      den2d[i, pl.ds(j * 16, 16)] = z16
    return ()
  lax.fori_loop(0, NR, _zero, ())

  for k in range(NPAD // NS // NR):
    pltpu.async_copy(asn_t, outacc.at[pl.ds((s * 8 + k) * NR, NR)],
                     gsems[k % 2])
  @pl.when(s < NR // 8)
  def _():
    pltpu.async_copy(asn_t.at[pl.ds(0, 8)], densh.at[pl.ds(s * 8, 8)],
                     gsems[2])
  for k in range(NPAD // NS // NR):
    pltpu.make_async_copy(asn_t, outacc.at[pl.ds((s * 8 + k) * NR, NR)],
                          gsems[k % 2]).wait()
  @pl.when(s < NR // 8)
  def _():
    pltpu.make_async_copy(asn_t.at[pl.ds(0, 8)], densh.at[pl.ds(s * 8, 8)],
                          gsems[2]).wait()

  pltpu.sync_copy(asn_h, asn_t)
  pltpu.sync_copy(adn_h, adn_t)

  ii = lax.iota(jnp.int32, 16)
  for i in range(NR // 16):
    iota_r[0, pl.ds(i * 16, 16)] = ii + (i * 16)

  plsc.subcore_barrier()

  def stage(g, t):
    off = base + g * CH
    pltpu.async_copy(src_h.at[pl.ds(off, CH)], sbuf.at[t], tsems[t])
    pltpu.async_copy(dst_h.at[pl.ds(off, CH)], dbuf.at[t], tsems[t])
    pltpu.async_copy(ae_h.at[pl.ds(off, CH)], aebuf.at[t], tsems[t])

  def wait_stage(g, t):
    off = base + g * CH
    pltpu.make_async_copy(src_h.at[pl.ds(off, CH)], sbuf.at[t], tsems[t]).wait()
    pltpu.make_async_copy(dst_h.at[pl.ds(off, CH)], dbuf.at[t], tsems[t]).wait()
    pltpu.make_async_copy(ae_h.at[pl.ds(off, CH)], aebuf.at[t], tsems[t]).wait()

  def gather(t, b):
    pltpu.async_copy(xs_h.at[sbuf.at[t]], rows.at[b], gsems[b])

  def wait_gather(t, b):
    pltpu.make_async_copy(xs_h.at[sbuf.at[t]], rows.at[b], gsems[b]).wait()

  def wait_scatter(t, b):
    pltpu.make_async_copy(rows.at[b], outacc.at[dbuf.at[t]], ssems[b]).wait()

  def ex_compute(t):
    for i in range(CH // 16):
      s16 = sbuf[t, pl.ds(i * 16, 16)]
      d16 = dbuf[t, pl.ds(i * 16, 16)]
      av = (plsc.load_gather(asn_t, [s16 >> 7, s16 & 127])
            + plsc.load_gather(adn_t, [d16 >> 7, d16 & 127])
            + aebuf[t, pl.ds(i * 16, 16)])
      av = jnp.maximum(av, 0.2 * av)
      ex = jnp.exp(av)
      exbuf[0, pl.ds(i * 16, 16)] = ex
      plsc.addupdate_scatter(den2d, [d16 >> 7, d16 & 127], ex)

  def scale_scatter(t, b):
    def _srow(i, _):
      ex16 = exbuf[0, pl.ds(i * 16, 16)]
      for k in range(16):
        cv = jnp.full((16,), ex16[k], jnp.float32)
        e = i * 16 + k
        for j in range(8):
          rows[b, e, pl.ds(j * 16, 16)] = rows[b, e, pl.ds(j * 16, 16)] * cv
      return ()
    lax.fori_loop(0, CH // 16, _srow, ())

    pltpu.async_copy(rows.at[b], outacc.at[dbuf.at[t]], ssems[b], add=True)

  for g0 in range(4):
    stage(g0, g0)
  wait_stage(0, 0)
  gather(0, 0)
  wait_stage(1, 1)
  gather(1, 1)

  def chunk_body(g, u):
    t = u % 6
    b = u % 3
    ex_compute(t)

    @pl.when(g >= 1)
    def _():
      wait_scatter((u - 1) % 6, (u - 1) % 3)

    @pl.when(g + 2 < NCHUNK)
    def _():
      wait_stage(g + 2, (u + 2) % 6)
      gather((u + 2) % 6, (u + 2) % 3)

    @pl.when(g + 4 < NCHUNK)
    def _():
      stage(g + 4, (u + 4) % 6)
    wait_gather(b, b)
    scale_scatter(t, b)

  def six(p, _):
    for u in range(6):
      chunk_body(6 * p + u, u)
    return ()
  lax.fori_loop(0, NCHUNK // 6, six, ())

  wait_scatter((NCHUNK - 1) % 6, (NCHUNK - 1) % 3)

  offt = base + NCHUNK * CH
  pltpu.sync_copy(src_h.at[pl.ds(offt, CHT)], stail.at[0])
  pltpu.sync_copy(dst_h.at[pl.ds(offt, CHT)], dtail.at[0])
  pltpu.sync_copy(ae_h.at[pl.ds(offt, CHT)], aetail.at[0])
  pltpu.async_copy(xs_h.at[stail.at[0]], rows.at[0, pl.ds(0, CHT)],
                   gsems[0]).wait()
  for i in range(CHT // 16):
    s16 = stail[0, pl.ds(i * 16, 16)]
    d16 = dtail[0, pl.ds(i * 16, 16)]
    av = (plsc.load_gather(asn_t, [s16 >> 7, s16 & 127])
          + plsc.load_gather(adn_t, [d16 >> 7, d16 & 127])
          + aetail[0, pl.ds(i * 16, 16)])
    av = jnp.maximum(av, 0.2 * av)
    ex = jnp.exp(av)
    plsc.addupdate_scatter(den2d, [d16 >> 7, d16 & 127], ex)
    for k in range(16):
      cv = jnp.full((16,), ex[k], jnp.float32)
      e = i * 16 + k
      for j in range(8):
        rows[0, e, pl.ds(j * 16, 16)] = rows[0, e, pl.ds(j * 16, 16)] * cv
  pltpu.async_copy(rows.at[0, pl.ds(0, CHT)], outacc.at[dtail.at[0]],
                   ssems[0], add=True).wait()

  plsc.subcore_barrier()

  pltpu.async_copy(den2d, densh.at[iota_r.at[0]], dsem, add=True).wait()
  plsc.subcore_barrier()

  rows_per_tile = NPAD // NS
  pltpu.sync_copy(outacc.at[pl.ds(s * rows_per_tile, rows_per_tile)],
                  outp_h.at[c, pl.ds(s * rows_per_tile, rows_per_tile)])
  @pl.when(s < NR // 8)
  def _():
    pltpu.sync_copy(densh.at[pl.ds(s * 8, 8)],
                    denp_h.at[c, pl.ds(s * 8, 8)])


@functools.cache
def _sc_edge_kernel():
  return pl.kernel(
    _sc_body,
    out_type=[
        jax.ShapeDtypeStruct((NC, NPAD, D), jnp.float32),
        jax.ShapeDtypeStruct((NC, NR, 128), jnp.float32),
    ],
    mesh=plsc.VectorSubcoreMesh(core_axis_name="c", subcore_axis_name="s",
                                num_cores=NC, num_subcores=NS),
    compiler_params=pltpu.CompilerParams(needs_layout_passes=False),
    scratch_types=[
        pltpu.VMEM((NR, 128), jnp.float32),
        pltpu.VMEM((NR, 128), jnp.float32),
        pltpu.VMEM((NR, 128), jnp.float32),
        pltpu.VMEM((6, CH), jnp.int32),
        pltpu.VMEM((6, CH), jnp.int32),
        pltpu.VMEM((6, CH), jnp.float32),
        pltpu.VMEM((1, CH), jnp.float32),
        pltpu.VMEM((3, CH, D), jnp.float32),
        pltpu.VMEM((1, NR), jnp.int32),
        pltpu.VMEM((1, CHT), jnp.int32),
        pltpu.VMEM((1, CHT), jnp.int32),
        pltpu.VMEM((1, CHT), jnp.float32),
        pltpu.VMEM_SHARED((NPAD, D), jnp.float32),
        pltpu.VMEM_SHARED((NR, 128), jnp.float32),
        pltpu.SemaphoreType.DMA,
        pltpu.SemaphoreType.DMA,
        pltpu.SemaphoreType.DMA,
        pltpu.SemaphoreType.DMA,
        pltpu.SemaphoreType.DMA,
        pltpu.SemaphoreType.DMA,
        pltpu.SemaphoreType.DMA,
        pltpu.SemaphoreType.DMA,
        pltpu.SemaphoreType.DMA,
        pltpu.SemaphoreType.DMA,
        pltpu.SemaphoreType.DMA,
        pltpu.SemaphoreType.DMA,
        pltpu.SemaphoreType.DMA,
    ],
  )


def _sc_edge(*args):
  return _sc_edge_kernel()(*args)


_NB = 10
_BN = NPAD // _NB
_BE = 32768
_EB = -(-E // _BE)


def _full(shape):
  return pl.BlockSpec(shape, lambda i: tuple(0 for _ in shape))


def _rows(bs, width=None):
  if width is None:
    return pl.BlockSpec((bs,), lambda i: (i,))
  return pl.BlockSpec((bs, width), lambda i: (i, 0))


def _pre_body(x_r, ws_r, as_r, wd_r, ad_r, xs_r, asn_r, adn_r):
  x = x_r[...]
  xs = jnp.dot(x, ws_r[...], preferred_element_type=jnp.float32)
  xs_r[...] = xs
  asn_r[...] = jnp.sum(xs * as_r[...], axis=1)
  xd = jnp.dot(x, wd_r[...], preferred_element_type=jnp.float32)
  adn_r[...] = jnp.sum(xd * ad_r[...], axis=1)


def _tc_pre(x, ws, a_s, wd, a_d):
  return pl.pallas_call(
      _pre_body,
      grid=(_NB,),
      in_specs=[_rows(_BN, D), _full((D, D)), _full((1, D)),
                _full((D, D)), _full((1, D))],
      out_specs=[_rows(_BN, D), _rows(_BN), _rows(_BN)],
      out_shape=[jax.ShapeDtypeStruct((NPAD, D), jnp.float32),
                 jax.ShapeDtypeStruct((NPAD,), jnp.float32),
                 jax.ShapeDtypeStruct((NPAD,), jnp.float32)],
  )(x, ws, a_s.reshape(1, D), wd, a_d.reshape(1, D))


def _ae_body(ea_r, we1_r, ae1_r, we2_r, ae2_r, o1_r, o2_r):
  ea = ea_r[...]
  v1 = jnp.sum(we1_r[...] * ae1_r[...], axis=1)
  v2 = jnp.sum(we2_r[...] * ae2_r[...], axis=1)
  o1_r[...] = jnp.sum(ea * v1[:, None], axis=0)
  o2_r[...] = jnp.sum(ea * v2[:, None], axis=0)


def _tc_ae(edge_attr_t, we1, ae1, we2, ae2):
  de = edge_attr_t.shape[0]
  return pl.pallas_call(
      _ae_body,
      grid=(_EB,),
      in_specs=[pl.BlockSpec((de, _BE), lambda i: (0, i)),
                _full((de, D)), _full((1, D)),
                _full((de, D)), _full((1, D))],
      out_specs=[_rows(_BE), _rows(_BE)],
      out_shape=[jax.ShapeDtypeStruct((E,), jnp.float32),
                 jax.ShapeDtypeStruct((E,), jnp.float32)],
  )(edge_attr_t, we1, ae1.reshape(1, D), we2, ae2.reshape(1, D))


def _mid_body(op_r, dp_r, x_r, wl_r, bl_r, b_r,
              ws2_r, as2_r, wd2_r, ad2_r,
              h_r, xs2_r, asn2_r, adn2_r):
  den = dp_r[0] + dp_r[1]
  gat = (op_r[0] + op_r[1]) * (1.0 / (den + 1e-16))[:, None] + b_r[...]
  lin = jnp.dot(x_r[...], wl_r[...], preferred_element_type=jnp.float32)
  h = jnp.maximum(gat + lin + bl_r[...], 0.0)
  h_r[...] = h
  xs2 = jnp.dot(h, ws2_r[...], preferred_element_type=jnp.float32)
  xs2_r[...] = xs2
  asn2_r[...] = jnp.sum(xs2 * as2_r[...], axis=1)
  xd2 = jnp.dot(h, wd2_r[...], preferred_element_type=jnp.float32)
  adn2_r[...] = jnp.sum(xd2 * ad2_r[...], axis=1)


def _tc_mid(outp, denp, x, wl, bl, b, ws2, as2, wd2, ad2):
  return pl.pallas_call(
      _mid_body,
      grid=(_NB,),
      in_specs=[pl.BlockSpec((NC, _BN, D), lambda i: (0, i, 0)),
                pl.BlockSpec((NC, _BN), lambda i: (0, i)),
                _rows(_BN, D), _full((D, D)), _full((1, D)), _full((1, D)),
                _full((D, D)), _full((1, D)), _full((D, D)), _full((1, D))],
      out_specs=[_rows(_BN, D), _rows(_BN, D), _rows(_BN), _rows(_BN)],
      out_shape=[jax.ShapeDtypeStruct((NPAD, D), jnp.float32),
                 jax.ShapeDtypeStruct((NPAD, D), jnp.float32),
                 jax.ShapeDtypeStruct((NPAD,), jnp.float32),
                 jax.ShapeDtypeStruct((NPAD,), jnp.float32)],
  )(outp, denp, x, wl, bl.reshape(1, D), b.reshape(1, D),
    ws2, as2.reshape(1, D), wd2, ad2.reshape(1, D))


def _post_body(op_r, dp_r, h_r, wl_r, bl_r, b_r, o_r):
  den = dp_r[0] + dp_r[1]
  gat = (op_r[0] + op_r[1]) * (1.0 / (den + 1e-16))[:, None] + b_r[...]
  lin = jnp.dot(h_r[...], wl_r[...], preferred_element_type=jnp.float32)
  o_r[...] = gat + lin + bl_r[...]


def _tc_post(outp, denp, h, wl, bl, b):
  return pl.pallas_call(
      _post_body,
      grid=(_NB,),
      in_specs=[pl.BlockSpec((NC, _BN, D), lambda i: (0, i, 0)),
                pl.BlockSpec((NC, _BN), lambda i: (0, i)),
                _rows(_BN, D), _full((D, D)), _full((1, D)), _full((1, D))],
      out_specs=_rows(_BN, D),
      out_shape=jax.ShapeDtypeStruct((N, D), jnp.float32),
  )(outp, denp, h, wl, bl.reshape(1, D), b.reshape(1, D))


@jax.jit
def kernel(x, edge_index, edge_attr,
           W_src1, W_dst1, att_src1, att_dst1, W_edge1, att_edge1, b1, Wl1, bl1,
           W_src2, W_dst2, att_src2, att_dst2, W_edge2, att_edge2, b2, Wl2, bl2):
  src = edge_index[0]
  dst = edge_index[1]

  ae1, ae2 = _tc_ae(edge_attr.T, W_edge1, att_edge1, W_edge2, att_edge2)

  xs1, asn1, adn1 = _tc_pre(x, W_src1, att_src1, W_dst1, att_dst1)
  outp1, denp1 = _sc_edge(xs1, asn1.reshape(NR, 128), adn1.reshape(NR, 128),
                          ae1, src, dst)
  h, xs2, asn2, adn2 = _tc_mid(
      outp1, denp1.reshape(NC, NPAD), x, Wl1, bl1, b1,
      W_src2, att_src2, W_dst2, att_dst2)

  outp2, denp2 = _sc_edge(xs2, asn2.reshape(NR, 128), adn2.reshape(NR, 128),
                          ae2, src, dst)
  return _tc_post(outp2, denp2.reshape(NC, NPAD), h, Wl2, bl2, b2)
